# baseline (speedup 1.0000x reference)
import functools

import numpy as np
import jax
import jax.numpy as jnp
from jax import lax
from jax.experimental import pallas as pl
from jax.experimental.pallas import tpu as pltpu
from jax.experimental.pallas import tpu_sc as plsc

_N_LEVELS = 16
_LOG2 = 19
_HASH = 1 << _LOG2
_MASK = _HASH - 1
_B = 524288
_FDIM = 2 * _N_LEVELS
_P1 = int(np.uint32(2654435761).view(np.int32))
_P2 = 805459861
_BFAC = np.exp((np.log(512.0) - np.log(16.0)) / (_N_LEVELS - 1))
_RES = np.floor(16.0 * _BFAC ** np.arange(_N_LEVELS)).astype(np.float32)
_GRID = [float(np.float32(2.0) / np.float32(r)) for r in _RES]
_UB = [float(np.float32(2.0) / np.float32(g)) for g in _GRID]

_LANES = 16
_C = 1024

_info = plsc.get_sparse_core_info()
_NC, _NS = _info.num_cores, _info.num_subcores
_NW = _NC * _NS
_PPW = _B // _NW
_NCHUNKS = _PPW // _C

_mesh = plsc.VectorSubcoreMesh(core_axis_name="c", subcore_axis_name="s")


def _loop_i32(n, body):
    def step(i, _):
        body(i)
        return i + np.int32(1), None

    lax.scan(step, np.int32(0), None, length=n)


@functools.partial(
    pl.kernel,
    out_type=jax.ShapeDtypeStruct((_B, _FDIM), jnp.float32),
    mesh=_mesh,
    scratch_types=[
        pltpu.VMEM((_C, 3), jnp.float32),
        pltpu.VMEM((8 * _C,), jnp.int32),
        pltpu.VMEM((8 * _C, 2), jnp.float32),
        pltpu.VMEM((_C,), jnp.float32),
        pltpu.VMEM((_C,), jnp.float32),
        pltpu.VMEM((_C,), jnp.float32),
        pltpu.VMEM((_C, _FDIM), jnp.float32),
        pltpu.SemaphoreType.DMA,
    ],
    compiler_params=pltpu.CompilerParams(
        needs_layout_passes=False, use_tc_tiling_on_sc=False
    ),
)
def _hash_embed(x_hbm, tab, out, x_v, idx_v, rows_v, wx_v, wy_v, wz_v, out_v, sem):
    i32 = jnp.int32
    wid = lax.axis_index("s") * i32(_NC) + lax.axis_index("c")
    base0 = wid * i32(_PPW)

    def chunk_body(ch):
        base = base0 + ch * i32(_C)
        pltpu.sync_copy(x_hbm.at[pl.ds(base, _C), :], x_v)

        for l in range(_N_LEVELS):
            grid = _GRID[l]
            ub = _UB[l]
            lbase = l * _HASH

            def hash_body(g, grid=grid, ub=ub, lbase=lbase):
                j0 = g * i32(_LANES)
                row = j0 + lax.iota(jnp.int32, _LANES)
                bl = []
                for d, wref in enumerate((wx_v, wy_v, wz_v)):
                    xd = plsc.load_gather(x_v, [row, jnp.full((_LANES,), d, jnp.int32)])
                    u = (xd + 1.0) / grid
                    t = jnp.minimum(jnp.maximum(u, 0.0), ub)
                    bi = t.astype(jnp.int32)
                    wref[pl.ds(j0, _LANES)] = u - bi.astype(jnp.float32)
                    bl.append(bi)
                bx, by, bz = bl
                hx = (bx, bx + i32(1))
                hy0 = by * i32(_P1)
                hz0 = bz * i32(_P2)
                hy = (hy0, hy0 + i32(_P1))
                hz = (hz0, hz0 + i32(_P2))
                c = 0
                for i in (0, 1):
                    for j in (0, 1):
                        exy = hx[i] ^ hy[j]
                        for k in (0, 1):
                            h = ((exy ^ hz[k]) & i32(_MASK)) + i32(lbase)
                            idx_v[pl.ds(i32(c * _C) + j0, _LANES)] = h
                            c += 1

            _loop_i32(_C // _LANES, hash_body)

            pltpu.async_copy(tab.at[idx_v], rows_v, sem).wait()

            def interp_body(g, l=l):
                j0 = g * i32(_LANES)
                row = j0 + lax.iota(jnp.int32, _LANES)
                wx = wx_v[pl.ds(j0, _LANES)]
                wy = wy_v[pl.ds(j0, _LANES)]
                wz = wz_v[pl.ds(j0, _LANES)]
                wxt = (1.0 - wx, wx)
                wyt = (1.0 - wy, wy)
                wzt = (1.0 - wz, wz)
                acc0 = jnp.zeros((_LANES,), jnp.float32)
                acc1 = jnp.zeros((_LANES,), jnp.float32)
                c = 0
                for i in (0, 1):
                    for j in (0, 1):
                        cxy = wxt[i] * wyt[j]
                        for k in (0, 1):
                            coeff = cxy * wzt[k]
                            r = i32(c * _C) + row
                            e0 = plsc.load_gather(rows_v, [r, jnp.zeros((_LANES,), jnp.int32)])
                            e1 = plsc.load_gather(rows_v, [r, jnp.full((_LANES,), 1, jnp.int32)])
                            acc0 = acc0 + coeff * e0
                            acc1 = acc1 + coeff * e1
                            c += 1
                col0 = jnp.full((_LANES,), 2 * l, jnp.int32)
                plsc.store_scatter(out_v, [row, col0], acc0)
                plsc.store_scatter(out_v, [row, col0 + i32(1)], acc1)

            _loop_i32(_C // _LANES, interp_body)

        pltpu.sync_copy(out_v, out.at[pl.ds(base, _C), :])

    _loop_i32(_NCHUNKS, chunk_body)


def kernel(x, tables):
    x = jnp.asarray(x, jnp.float32)
    one = lax.optimization_barrier(jnp.float32(1.0))
    tab = jnp.asarray(tables, jnp.float32).reshape(_N_LEVELS * _HASH, 2) * one
    return _hash_embed(x, tab)


# speedup vs baseline: 2.3622x; 2.3622x over previous
import functools

import numpy as np
import jax
import jax.numpy as jnp
from jax import lax
from jax.experimental import pallas as pl
from jax.experimental.pallas import tpu as pltpu
from jax.experimental.pallas import tpu_sc as plsc

_N_LEVELS = 16
_LOG2 = 19
_HASH = 1 << _LOG2
_MASK = _HASH - 1
_B = 524288
_FDIM = 2 * _N_LEVELS
_P1 = int(np.uint32(2654435761).view(np.int32))
_P2 = 805459861
_BFAC = np.exp((np.log(512.0) - np.log(16.0)) / (_N_LEVELS - 1))
_RES = np.floor(16.0 * _BFAC ** np.arange(_N_LEVELS)).astype(np.float32)
_GRID = [float(np.float32(2.0) / np.float32(r)) for r in _RES]
_UB = [float(np.float32(2.0) / np.float32(g)) for g in _GRID]

_LANES = 16
_C = 1024

_info = plsc.get_sparse_core_info()
_NC, _NS = _info.num_cores, _info.num_subcores
_NW = _NC * _NS
_PPW = _B // _NW
_NCHUNKS = _PPW // _C

_mesh = plsc.VectorSubcoreMesh(core_axis_name="c", subcore_axis_name="s")


def _loop_i32(n, body):
    def step(i, _):
        body(i)
        return i + np.int32(1), None

    lax.scan(step, np.int32(0), None, length=n)


@functools.partial(
    pl.kernel,
    out_type=jax.ShapeDtypeStruct((_B, _FDIM), jnp.float32),
    mesh=_mesh,
    scratch_types=[
        pltpu.VMEM((_C, 3), jnp.float32),
        pltpu.VMEM((16 * _C,), jnp.int32),
        pltpu.VMEM((16 * _C,), jnp.float32),
        pltpu.VMEM((_C,), jnp.float32),
        pltpu.VMEM((_C,), jnp.float32),
        pltpu.VMEM((_C,), jnp.float32),
        pltpu.VMEM((_C, _FDIM), jnp.float32),
        pltpu.SemaphoreType.DMA,
    ],
    compiler_params=pltpu.CompilerParams(
        needs_layout_passes=False, use_tc_tiling_on_sc=False
    ),
)
def _hash_embed(x_hbm, tab, out, x_v, idx_v, rows_v, wx_v, wy_v, wz_v, out_v, sem):
    i32 = jnp.int32
    wid = lax.axis_index("s") * i32(_NC) + lax.axis_index("c")
    base0 = wid * i32(_PPW)

    def chunk_body(ch):
        base = base0 + ch * i32(_C)
        pltpu.sync_copy(x_hbm.at[pl.ds(base, _C), :], x_v)

        for l in range(_N_LEVELS):
            grid = _GRID[l]
            ub = _UB[l]
            lbase2 = l << 20

            def hash_body(g, grid=grid, ub=ub, lbase2=lbase2):
                j0 = g * i32(_LANES)
                row = j0 + lax.iota(jnp.int32, _LANES)
                bl = []
                for d, wref in enumerate((wx_v, wy_v, wz_v)):
                    xd = plsc.load_gather(x_v, [row, jnp.full((_LANES,), d, jnp.int32)])
                    u = (xd + 1.0) / grid
                    t = jnp.minimum(jnp.maximum(u, 0.0), ub)
                    bi = t.astype(jnp.int32)
                    wref[pl.ds(j0, _LANES)] = u - bi.astype(jnp.float32)
                    bl.append(bi)
                bx, by, bz = bl
                hx = (bx, bx + i32(1))
                hy0 = by * i32(_P1)
                hz0 = bz * i32(_P2)
                hy = (hy0, hy0 + i32(_P1))
                hz = (hz0, hz0 + i32(_P2))
                c = 0
                for i in (0, 1):
                    for j in (0, 1):
                        exy = hx[i] ^ hy[j]
                        for k in (0, 1):
                            h = (exy ^ hz[k]) & i32(_MASK)
                            e0 = i32(lbase2) + ((h & i32(~127)) << 1) + (h & i32(127))
                            idx_v[pl.ds(i32(2 * c * _C) + j0, _LANES)] = e0
                            idx_v[pl.ds(i32((2 * c + 1) * _C) + j0, _LANES)] = e0 + i32(128)
                            c += 1

            _loop_i32(_C // _LANES, hash_body)

            pltpu.async_copy(tab.at[idx_v], rows_v, sem).wait()

            def interp_body(g, l=l):
                j0 = g * i32(_LANES)
                row = j0 + lax.iota(jnp.int32, _LANES)
                wx = wx_v[pl.ds(j0, _LANES)]
                wy = wy_v[pl.ds(j0, _LANES)]
                wz = wz_v[pl.ds(j0, _LANES)]
                wxt = (1.0 - wx, wx)
                wyt = (1.0 - wy, wy)
                wzt = (1.0 - wz, wz)
                acc0 = jnp.zeros((_LANES,), jnp.float32)
                acc1 = jnp.zeros((_LANES,), jnp.float32)
                c = 0
                for i in (0, 1):
                    for j in (0, 1):
                        cxy = wxt[i] * wyt[j]
                        for k in (0, 1):
                            coeff = cxy * wzt[k]
                            e0 = plsc.load_gather(rows_v, [i32(2 * c * _C) + row])
                            e1 = plsc.load_gather(rows_v, [i32((2 * c + 1) * _C) + row])
                            acc0 = acc0 + coeff * e0
                            acc1 = acc1 + coeff * e1
                            c += 1
                col0 = jnp.full((_LANES,), 2 * l, jnp.int32)
                plsc.store_scatter(out_v, [row, col0], acc0)
                plsc.store_scatter(out_v, [row, col0 + i32(1)], acc1)

            _loop_i32(_C // _LANES, interp_body)

        pltpu.sync_copy(out_v, out.at[pl.ds(base, _C), :])

    _loop_i32(_NCHUNKS, chunk_body)


def kernel(x, tables):
    x = jnp.asarray(x, jnp.float32)
    tab = (
        jnp.asarray(tables, jnp.float32)
        .reshape(_N_LEVELS, _HASH // 128, 128, 2)
        .transpose(0, 1, 3, 2)
        .reshape(_N_LEVELS * _HASH * 2)
    )
    return _hash_embed(x, tab)


# speedup vs baseline: 3.0870x; 1.3068x over previous
import functools

import numpy as np
import jax
import jax.numpy as jnp
from jax import lax
from jax.experimental import pallas as pl
from jax.experimental.pallas import tpu as pltpu
from jax.experimental.pallas import tpu_sc as plsc

_N_LEVELS = 16
_LOG2 = 19
_HASH = 1 << _LOG2
_MASK = _HASH - 1
_B = 524288
_FDIM = 2 * _N_LEVELS
_P1 = int(np.uint32(2654435761).view(np.int32))
_P2 = 805459861
_BFAC = np.exp((np.log(512.0) - np.log(16.0)) / (_N_LEVELS - 1))
_RES = np.floor(16.0 * _BFAC ** np.arange(_N_LEVELS)).astype(np.float32)
_GRID = [float(np.float32(2.0) / np.float32(r)) for r in _RES]
_UB = [float(np.float32(2.0) / np.float32(g)) for g in _GRID]

_LANES = 16
_C = 1024
_STG = 2048
_TW = _N_LEVELS * _HASH * 2
_TILE_WORDS = _TW // 16

_info = plsc.get_sparse_core_info()
_NC, _NS = _info.num_cores, _info.num_subcores
_NW = _NC * _NS
_PPW = _B // _NW
_NCHUNKS = _PPW // _C

_mesh = plsc.VectorSubcoreMesh(core_axis_name="c", subcore_axis_name="s")


def _loop_i32(n, body):
    def step(i, _):
        body(i)
        return i + np.int32(1), None

    lax.scan(step, np.int32(0), None, length=n)


@functools.partial(
    pl.kernel,
    out_type=(
        jax.ShapeDtypeStruct((_B, _FDIM), jnp.float32),
        jax.ShapeDtypeStruct((_N_LEVELS * _HASH, 2), jnp.float32),
    ),
    mesh=_mesh,
    scratch_types=[
        pltpu.VMEM((3 * _C,), jnp.float32),
        pltpu.VMEM((_STG,), jnp.float32),
        pltpu.VMEM((_STG // 2, 2), jnp.float32),
        pltpu.VMEM((8 * _C,), jnp.int32),
        pltpu.VMEM((8 * _C, 2), jnp.float32),
        pltpu.VMEM((_C,), jnp.float32),
        pltpu.VMEM((_C,), jnp.float32),
        pltpu.VMEM((_C,), jnp.float32),
        pltpu.VMEM((_C, _FDIM), jnp.float32),
        pltpu.SemaphoreType.DMA,
    ],
    compiler_params=pltpu.CompilerParams(
        needs_layout_passes=False, use_tc_tiling_on_sc=False
    ),
)
def _hash_embed(
    x_hbm, tab, out, rm_tab,
    x_v, stg_v, il_v, idx_v, rows_v, wx_v, wy_v, wz_v, out_v, sem,
):
    i32 = jnp.int32
    sid = lax.axis_index("s")
    wid = sid * i32(_NC) + lax.axis_index("c")
    base0 = wid * i32(_PPW)

    def stage_body(s):
        src = sid * i32(_TILE_WORDS) + s * i32(_STG)
        pltpu.sync_copy(tab.at[pl.ds(src, _STG)], stg_v)

        def il_body(g):
            j0 = g * i32(_LANES)
            blk = j0 >> i32(7)
            off = j0 & i32(127)
            tile = blk >> i32(1)
            feat = blk & i32(1)
            lh = (tile << i32(7)) + off
            fcol = jnp.zeros((_LANES,), jnp.int32) + feat
            v = stg_v[pl.ds(j0, _LANES)]
            plsc.store_scatter(il_v, [lh + lax.iota(jnp.int32, _LANES), fcol], v)

        _loop_i32(_STG // _LANES, il_body)

        dst_row = (sid * i32(_TILE_WORDS) + s * i32(_STG)) >> i32(1)
        pltpu.sync_copy(il_v, rm_tab.at[pl.ds(dst_row, _STG // 2), :])

    _loop_i32(_TILE_WORDS // _STG, stage_body)
    plsc.subcore_barrier()

    def chunk_body(ch):
        base = base0 + ch * i32(_C)
        pltpu.sync_copy(x_hbm.at[pl.ds(base * i32(3), 3 * _C)], x_v)

        for l in range(_N_LEVELS):
            grid = _GRID[l]
            ub = _UB[l]
            lbase = l * _HASH

            def hash_body(g, grid=grid, ub=ub, lbase=lbase):
                j0 = g * i32(_LANES)
                row = j0 + lax.iota(jnp.int32, _LANES)
                bl = []
                row3 = row + row + row
                for d, wref in enumerate((wx_v, wy_v, wz_v)):
                    xd = plsc.load_gather(x_v, [row3 + i32(d)])
                    u = (xd + 1.0) / grid
                    t = jnp.minimum(jnp.maximum(u, 0.0), ub)
                    bi = t.astype(jnp.int32)
                    wref[pl.ds(j0, _LANES)] = u - bi.astype(jnp.float32)
                    bl.append(bi)
                bx, by, bz = bl
                hx = (bx, bx + i32(1))
                hy0 = by * i32(_P1)
                hz0 = bz * i32(_P2)
                hy = (hy0, hy0 + i32(_P1))
                hz = (hz0, hz0 + i32(_P2))
                c = 0
                for i in (0, 1):
                    for j in (0, 1):
                        exy = hx[i] ^ hy[j]
                        for k in (0, 1):
                            h = ((exy ^ hz[k]) & i32(_MASK)) + i32(lbase)
                            idx_v[pl.ds(i32(c * _C) + j0, _LANES)] = h
                            c += 1

            _loop_i32(_C // _LANES, hash_body)

            pltpu.async_copy(rm_tab.at[idx_v], rows_v, sem).wait()

            def interp_body(g, l=l):
                j0 = g * i32(_LANES)
                row = j0 + lax.iota(jnp.int32, _LANES)
                wx = wx_v[pl.ds(j0, _LANES)]
                wy = wy_v[pl.ds(j0, _LANES)]
                wz = wz_v[pl.ds(j0, _LANES)]
                wxt = (1.0 - wx, wx)
                wyt = (1.0 - wy, wy)
                wzt = (1.0 - wz, wz)
                acc0 = jnp.zeros((_LANES,), jnp.float32)
                acc1 = jnp.zeros((_LANES,), jnp.float32)
                c = 0
                for i in (0, 1):
                    for j in (0, 1):
                        cxy = wxt[i] * wyt[j]
                        for k in (0, 1):
                            coeff = cxy * wzt[k]
                            r = i32(c * _C) + row
                            e0 = plsc.load_gather(
                                rows_v, [r, jnp.zeros((_LANES,), jnp.int32)]
                            )
                            e1 = plsc.load_gather(
                                rows_v, [r, jnp.full((_LANES,), 1, jnp.int32)]
                            )
                            acc0 = acc0 + coeff * e0
                            acc1 = acc1 + coeff * e1
                            c += 1
                col0 = jnp.full((_LANES,), 2 * l, jnp.int32)
                plsc.store_scatter(out_v, [row, col0], acc0)
                plsc.store_scatter(out_v, [row, col0 + i32(1)], acc1)

            _loop_i32(_C // _LANES, interp_body)

        pltpu.sync_copy(out_v, out.at[pl.ds(base, _C), :])

    _loop_i32(_NCHUNKS, chunk_body)


def kernel(x, tables):
    x = jnp.asarray(x, jnp.float32).reshape(3 * _B)
    tab = (
        jnp.asarray(tables, jnp.float32)
        .reshape(_N_LEVELS, _HASH // 128, 128, 2)
        .transpose(0, 1, 3, 2)
        .reshape(_N_LEVELS * _HASH * 2)
    )
    out, _ = _hash_embed(x, tab)
    return out


# speedup vs baseline: 4.2536x; 1.3779x over previous
import functools

import numpy as np
import jax
import jax.numpy as jnp
from jax import lax
from jax.experimental import pallas as pl
from jax.experimental.pallas import tpu as pltpu
from jax.experimental.pallas import tpu_sc as plsc

_N_LEVELS = 16
_LOG2 = 19
_HASH = 1 << _LOG2
_MASK = _HASH - 1
_B = 524288
_FDIM = 2 * _N_LEVELS
_P1 = int(np.uint32(2654435761).view(np.int32))
_P2 = 805459861
_BFAC = np.exp((np.log(512.0) - np.log(16.0)) / (_N_LEVELS - 1))
_RES = np.floor(16.0 * _BFAC ** np.arange(_N_LEVELS)).astype(np.float32)
_GRID = [float(np.float32(2.0) / np.float32(r)) for r in _RES]
_UB = [float(np.float32(2.0) / np.float32(g)) for g in _GRID]

_LANES = 16
_C = 1024
_STG = 2048
_TW = _N_LEVELS * _HASH * 2
_TILE_WORDS = _TW // 16

_info = plsc.get_sparse_core_info()
_NC, _NS = _info.num_cores, _info.num_subcores
_NW = _NC * _NS
_PPW = _B // _NW
_NCHUNKS = _PPW // _C

_mesh = plsc.VectorSubcoreMesh(core_axis_name="c", subcore_axis_name="s")


def _loop_i32(n, body):
    def step(i, _):
        body(i)
        return i + np.int32(1), None

    lax.scan(step, np.int32(0), None, length=n)


@functools.partial(
    pl.kernel,
    out_type=(
        jax.ShapeDtypeStruct((_B * _FDIM,), jnp.float32),
        jax.ShapeDtypeStruct((_N_LEVELS * _HASH, 2), jnp.float32),
    ),
    mesh=_mesh,
    scratch_types=[
        pltpu.VMEM((3 * _C,), jnp.float32),
        pltpu.VMEM((_STG,), jnp.float32),
        pltpu.VMEM((_STG // 2, 2), jnp.float32),
        pltpu.VMEM((8 * _C,), jnp.int32),
        pltpu.VMEM((8 * _C, 2), jnp.float32),
        pltpu.VMEM((_C,), jnp.float32),
        pltpu.VMEM((_C,), jnp.float32),
        pltpu.VMEM((_C,), jnp.float32),
        pltpu.VMEM((_FDIM * _C,), jnp.float32),
        pltpu.SemaphoreType.DMA,
        pltpu.SemaphoreType.DMA,
    ],
    compiler_params=pltpu.CompilerParams(
        needs_layout_passes=False, use_tc_tiling_on_sc=False
    ),
)
def _hash_embed(
    x_hbm, tab, out, rm_tab,
    x_v, stg_v, il_v, idx_v, rows_v, wx_v, wy_v, wz_v, out_v, sem, sem2,
):
    i32 = jnp.int32
    sid = lax.axis_index("s")
    wid = sid * i32(_NC) + lax.axis_index("c")
    base0 = wid * i32(_PPW)

    def stage_body(s):
        src = sid * i32(_TILE_WORDS) + s * i32(_STG)
        pltpu.sync_copy(tab.at[pl.ds(src, _STG)], stg_v)

        def il_body(g):
            j0 = g * i32(_LANES)
            blk = j0 >> i32(7)
            off = j0 & i32(127)
            tile = blk >> i32(1)
            feat = blk & i32(1)
            lh = (tile << i32(7)) + off
            fcol = jnp.zeros((_LANES,), jnp.int32) + feat
            v = stg_v[pl.ds(j0, _LANES)]
            plsc.store_scatter(il_v, [lh + lax.iota(jnp.int32, _LANES), fcol], v)

        _loop_i32(_STG // _LANES, il_body)

        dst_row = (sid * i32(_TILE_WORDS) + s * i32(_STG)) >> i32(1)
        pltpu.sync_copy(il_v, rm_tab.at[pl.ds(dst_row, _STG // 2), :])

    _loop_i32(_TILE_WORDS // _STG, stage_body)
    plsc.subcore_barrier()

    _H = _C // 2

    def chunk_body(ch):
        base = base0 + ch * i32(_C)
        b7_0 = base >> i32(7)
        pltpu.sync_copy(x_hbm.at[pl.ds(base * i32(3), 3 * _C)], x_v)

        for l in range(_N_LEVELS):
            grid = _GRID[l]
            ub = _UB[l]
            lbase = l * _HASH

            def hash_half(h, grid=grid, ub=ub, lbase=lbase):
                def hash_body(g):
                    jl = g * i32(_LANES)
                    j0 = i32(h * _H) + jl
                    row = j0 + lax.iota(jnp.int32, _LANES)
                    bl = []
                    row3 = row + row + row
                    for d, wref in enumerate((wx_v, wy_v, wz_v)):
                        xd = plsc.load_gather(x_v, [row3 + i32(d)])
                        u = (xd + 1.0) / grid
                        t = jnp.minimum(jnp.maximum(u, 0.0), ub)
                        bi = t.astype(jnp.int32)
                        wref[pl.ds(j0, _LANES)] = u - bi.astype(jnp.float32)
                        bl.append(bi)
                    bx, by, bz = bl
                    hx = (bx, bx + i32(1))
                    hy0 = by * i32(_P1)
                    hz0 = bz * i32(_P2)
                    hy = (hy0, hy0 + i32(_P1))
                    hz = (hz0, hz0 + i32(_P2))
                    c = 0
                    for i in (0, 1):
                        for j in (0, 1):
                            exy = hx[i] ^ hy[j]
                            for k in (0, 1):
                                hh = ((exy ^ hz[k]) & i32(_MASK)) + i32(lbase)
                                idx_v[pl.ds(i32(h * 8 * _H + c * _H) + jl, _LANES)] = hh
                                c += 1

                _loop_i32(_H // _LANES, hash_body)

            def interp_half(h, l=l):
                def interp_body(g):
                    jl = g * i32(_LANES)
                    j0 = i32(h * _H) + jl
                    row = j0 + lax.iota(jnp.int32, _LANES)
                    wx = wx_v[pl.ds(j0, _LANES)]
                    wy = wy_v[pl.ds(j0, _LANES)]
                    wz = wz_v[pl.ds(j0, _LANES)]
                    wxt = (1.0 - wx, wx)
                    wyt = (1.0 - wy, wy)
                    wzt = (1.0 - wz, wz)
                    acc0 = jnp.zeros((_LANES,), jnp.float32)
                    acc1 = jnp.zeros((_LANES,), jnp.float32)
                    c = 0
                    for i in (0, 1):
                        for j in (0, 1):
                            cxy = wxt[i] * wyt[j]
                            for k in (0, 1):
                                coeff = cxy * wzt[k]
                                r = i32(h * 8 * _H + c * _H) + jl + lax.iota(jnp.int32, _LANES)
                                e0 = plsc.load_gather(
                                    rows_v, [r, jnp.zeros((_LANES,), jnp.int32)]
                                )
                                e1 = plsc.load_gather(
                                    rows_v, [r, jnp.full((_LANES,), 1, jnp.int32)]
                                )
                                acc0 = acc0 + coeff * e0
                                acc1 = acc1 + coeff * e1
                                c += 1
                    pvec = ((row >> i32(7)) << i32(10)) + (row & i32(127))
                    f = 2 * l
                    plsc.store_scatter(
                        out_v, [pvec + i32((f >> 3) * 8192 + (f & 7) * 128)], acc0
                    )
                    plsc.store_scatter(
                        out_v, [pvec + i32((f >> 3) * 8192 + ((f + 1) & 7) * 128)], acc1
                    )

                _loop_i32(_H // _LANES, interp_body)

            hash_half(0)
            cpA = pltpu.async_copy(
                rm_tab.at[idx_v.at[pl.ds(0, 8 * _H)]],
                rows_v.at[pl.ds(0, 8 * _H), :], sem,
            )
            hash_half(1)
            cpB = pltpu.async_copy(
                rm_tab.at[idx_v.at[pl.ds(8 * _H, 8 * _H)]],
                rows_v.at[pl.ds(8 * _H, 8 * _H), :], sem2,
            )
            cpA.wait()
            interp_half(0)
            cpB.wait()
            interp_half(1)

        for f3 in range(4):
            dst = (i32(f3 * 4096) + b7_0) * i32(1024)
            pltpu.sync_copy(
                out_v.at[pl.ds(f3 * 8192, 8192)], out.at[pl.ds(dst, 8192)]
            )

    _loop_i32(_NCHUNKS, chunk_body)


def kernel(x, tables):
    x = jnp.asarray(x, jnp.float32).reshape(3 * _B)
    tab = (
        jnp.asarray(tables, jnp.float32)
        .reshape(_N_LEVELS, _HASH // 128, 128, 2)
        .transpose(0, 1, 3, 2)
        .reshape(_N_LEVELS * _HASH * 2)
    )
    raw, _ = _hash_embed(x, tab)
    return (
        raw.reshape(4, _B // 128, 8, 128)
        .transpose(1, 3, 0, 2)
        .reshape(_B, _FDIM)
    )
